# Initial kernel scaffold; baseline (speedup 1.0000x reference)
import jax
import jax.numpy as jnp
from jax.experimental import pallas as pl


def kernel(node_features, incidence_matrix, W, b, epsilon):
    raise NotImplementedError("write your pallas kernel here")


# speedup vs baseline: 696.6153x; 696.6153x over previous
import jax
import jax.numpy as jnp
from jax import lax
from jax.experimental import pallas as pl


def _hgmp_kernel(x_ref, inc_ref, w_ref, b_ref, eps_ref, out_ref):
    x = x_ref[...]
    mask = (inc_ref[...] > 0).astype(jnp.float32)

    sums = lax.dot_general(mask, x, (((0,), (0,)), ((), ())),
                           preferred_element_type=jnp.float32)
    ones = jnp.ones((x.shape[0], 1), dtype=jnp.float32)
    counts = lax.dot_general(mask, ones, (((0,), (0,)), ((), ())),
                             preferred_element_type=jnp.float32)
    vf = sums / jnp.maximum(counts, 1.0)

    svf = lax.dot_general(mask, vf, (((1,), (0,)), ((), ())),
                          preferred_element_type=jnp.float32)

    combined = (1.0 + eps_ref[0, 0]) * x + svf
    y = lax.dot_general(combined, w_ref[...], (((1,), (1,)), ((), ())),
                        preferred_element_type=jnp.float32) + b_ref[...]
    out_ref[...] = jnp.where(y > 0, y, 0.2 * y)


def kernel(node_features, incidence_matrix, W, b, epsilon):
    N, D = node_features.shape
    b2 = b.reshape(1, D)
    eps2 = epsilon.reshape(1, 1)
    return pl.pallas_call(
        _hgmp_kernel,
        out_shape=jax.ShapeDtypeStruct((N, D), jnp.float32),
    )(node_features, incidence_matrix, W, b2, eps2)
